# baseline (speedup 1.0000x reference)
import functools

import jax
import jax.numpy as jnp
from jax import lax
from jax.experimental import pallas as pl
from jax.experimental.pallas import tpu as pltpu


def _round_up(a, m):
    return (a + m - 1) // m * m


def _cnn_mxu_kernel(w2_ref, bias_ref, x_ref, out_ref, *, n_out, lout):
    L, C, TB = x_ref.shape
    x2 = x_ref[...].reshape(L * C, TB).astype(jnp.bfloat16)
    r = lax.dot_general(
        w2_ref[...], x2,
        dimension_numbers=(((0,), (0,)), ((), ())),
        preferred_element_type=jnp.float32,
    )

    n_pow2 = 1
    while n_pow2 * 2 <= lout:
        n_pow2 *= 2
    p = n_pow2 * n_out
    m = r[:p]
    while p > n_out:
        p //= 2
        m = jnp.maximum(m[:p], m[p:2 * p])
    for k in range(n_pow2, lout):
        m = jnp.maximum(m, r[k * n_out:(k + 1) * n_out])

    out_ref[...] = jnp.maximum(m + bias_ref[...], 0.0)[:, None, :]


def _build_w2(weight, lout):
    import numpy as np
    n_out, n_in, ksize = weight.shape
    length = lout + ksize - 1
    e = np.zeros((ksize, length, lout), dtype=np.float32)
    for j in range(ksize):
        for t in range(lout):
            e[j, t + j, t] = 1.0
    w2 = jnp.einsum('ocj,jlt->lcto', weight.astype(jnp.bfloat16),
                    jnp.asarray(e, dtype=jnp.bfloat16))
    return w2.reshape(length * n_in, lout * n_out)


@functools.partial(jax.jit, static_argnames=("block_b",))
def _forward(x, weight, bias, *, block_b=32768):
    B, C, L = x.shape
    O, Cw, K = weight.shape
    assert Cw == C and L >= K, (x.shape, weight.shape)
    lout = L - K + 1

    xt = jnp.transpose(x, (2, 1, 0))
    w2 = _build_w2(weight, lout)
    bias_col = bias.reshape(O, 1).astype(jnp.float32)

    tb = min(block_b, _round_up(B, 128))
    b_pad = _round_up(B, tb)
    if b_pad != B:
        xt = jnp.pad(xt, ((0, 0), (0, 0), (0, b_pad - B)))
    grid = b_pad // tb

    kernel_fn = functools.partial(_cnn_mxu_kernel, n_out=O, lout=lout)

    in_bytes = xt.size * xt.dtype.itemsize + w2.size * 2 + bias_col.size * 4
    out_bytes = O * b_pad * 4

    out = pl.pallas_call(
        kernel_fn,
        out_shape=jax.ShapeDtypeStruct((O, 1, b_pad), jnp.float32),
        grid=(grid,),
        in_specs=[
            pl.BlockSpec((L * C, lout * O), lambda i: (0, 0)),
            pl.BlockSpec((O, 1), lambda i: (0, 0)),
            pl.BlockSpec((L, C, tb), lambda i: (0, 0, i)),
        ],
        out_specs=pl.BlockSpec((O, 1, tb), lambda i: (0, 0, i)),
        compiler_params=pltpu.CompilerParams(
            dimension_semantics=("parallel",)),
        cost_estimate=pl.CostEstimate(
            flops=2 * b_pad * lout * C * K * O,
            transcendentals=0,
            bytes_accessed=in_bytes + out_bytes),
    )(w2, bias_col, xt)

    return jnp.transpose(out, (2, 0, 1))[:B]


def kernel(x, weight, bias):
    return _forward(x, weight, bias)
